# baseline (speedup 1.0000x reference)
import functools

import jax
import jax.numpy as jnp
from jax import lax
from jax.experimental import pallas as pl
from jax.experimental.pallas import tpu as pltpu
from jax.experimental.pallas import tpu_sc as plsc

NUM_GROUP = 1000000
EMBED_DIM = 32
BATCH = 16384

_NC = 2
_NS = 16
_NW = _NC * _NS
_B_PER_W = BATCH // _NW
_CHUNK = 128
_NCHUNK = _B_PER_W // _CHUNK


def _sc_gather(idx_hbm, table_hbm, out_hbm, idx_v, rows_v, sem):
    wid = lax.axis_index("s") * _NC + lax.axis_index("c")
    base = wid * _B_PER_W
    pltpu.sync_copy(idx_hbm.at[pl.ds(base, _B_PER_W)], idx_v)
    copies = []
    for j in range(_NCHUNK):
        copies.append(
            pltpu.async_copy(
                table_hbm.at[idx_v.at[pl.ds(j * _CHUNK, _CHUNK)]],
                rows_v.at[pl.ds(j * _CHUNK, _CHUNK), :],
                sem,
            )
        )
    for c in copies:
        c.wait()
    pltpu.sync_copy(rows_v, out_hbm.at[pl.ds(base, _B_PER_W)])


def kernel(num_group, table):
    mesh = plsc.VectorSubcoreMesh(core_axis_name="c", subcore_axis_name="s")
    run = functools.partial(
        pl.kernel,
        mesh=mesh,
        out_type=jax.ShapeDtypeStruct((BATCH, EMBED_DIM), jnp.float32),
        scratch_types=[
            pltpu.VMEM((_B_PER_W,), jnp.int32),
            pltpu.VMEM((_B_PER_W, EMBED_DIM), jnp.float32),
            pltpu.SemaphoreType.DMA,
        ],
        compiler_params=pltpu.CompilerParams(use_tc_tiling_on_sc=False),
    )(_sc_gather)
    return run(num_group.astype(jnp.int32), table)


# speedup vs baseline: 4.6365x; 4.6365x over previous
import functools

import jax
import jax.numpy as jnp
from jax import lax
from jax.experimental import pallas as pl
from jax.experimental.pallas import tpu as pltpu
from jax.experimental.pallas import tpu_sc as plsc

NUM_GROUP = 1000000
EMBED_DIM = 32
BATCH = 16384

_NC = 2
_NS = 16
_NW = _NC * _NS
_B_PER_W = BATCH // _NW
_NBUF = 8
_LANES = 128


def _idx_scalar(idx_v, b):
    cb = pl.multiple_of((b // 16) * 16, 16)
    chunk = idx_v[pl.ds(cb, 16)]
    sel = jnp.where(lax.iota(jnp.int32, 16) == b % 16, chunk, 0)
    return jnp.sum(sel)


def _issue_fetch(tab_hbm, stage_v, sem, i, buf):
    blk = pl.multiple_of((i // _LANES) * _LANES, _LANES)
    pltpu.async_copy(
        tab_hbm.at[:, pl.ds(blk, _LANES)], stage_v.at[buf], sem
    )


def _sc_gather(idx_hbm, tab_hbm, out_hbm, idx_v, stage_v, slab_v, sem):
    wid = lax.axis_index("s") * _NC + lax.axis_index("c")
    base = wid * _B_PER_W
    pltpu.sync_copy(idx_hbm.at[pl.ds(base, _B_PER_W)], idx_v)

    rows_lo = lax.iota(jnp.int32, 16)
    rows_hi = rows_lo + 16

    for b in range(_NBUF):
        _issue_fetch(tab_hbm, stage_v, sem, _idx_scalar(idx_v, b), b)

    def body(b, carry):
        buf = lax.rem(b, _NBUF)
        pltpu.make_async_copy(
            tab_hbm.at[:, pl.ds(0, _LANES)], stage_v.at[buf], sem
        ).wait()
        i = _idx_scalar(idx_v, b)
        lane = jnp.full((16,), i % _LANES, jnp.int32)
        col = jnp.full((16,), b, jnp.int32)
        lo = plsc.load_gather(stage_v.at[buf], [rows_lo, lane])
        hi = plsc.load_gather(stage_v.at[buf], [rows_hi, lane])
        plsc.store_scatter(slab_v, [rows_lo, col], lo)
        plsc.store_scatter(slab_v, [rows_hi, col], hi)

        @pl.when(b + _NBUF < _B_PER_W)
        def _():
            _issue_fetch(tab_hbm, stage_v, sem, _idx_scalar(idx_v, b + _NBUF), buf)

        return carry

    lax.fori_loop(0, _B_PER_W, body, 0, unroll=2)
    pltpu.sync_copy(slab_v, out_hbm.at[:, pl.ds(base, _B_PER_W)])


def kernel(num_group, table):
    mesh = plsc.VectorSubcoreMesh(core_axis_name="c", subcore_axis_name="s")
    run = functools.partial(
        pl.kernel,
        mesh=mesh,
        out_type=jax.ShapeDtypeStruct((EMBED_DIM, BATCH), jnp.float32),
        scratch_types=[
            pltpu.VMEM((_B_PER_W,), jnp.int32),
            pltpu.VMEM((_NBUF, EMBED_DIM, _LANES), jnp.float32),
            pltpu.VMEM((EMBED_DIM, _B_PER_W), jnp.float32),
            pltpu.SemaphoreType.DMA,
        ],
        compiler_params=pltpu.CompilerParams(needs_layout_passes=False),
    )(_sc_gather)
    out_t = run(num_group.astype(jnp.int32), table.T)
    return out_t.T


# speedup vs baseline: 4.6964x; 1.0129x over previous
import functools

import jax
import jax.numpy as jnp
from jax import lax
from jax.experimental import pallas as pl
from jax.experimental.pallas import tpu as pltpu
from jax.experimental.pallas import tpu_sc as plsc

NUM_GROUP = 1000000
EMBED_DIM = 32
BATCH = 16384

_NC = 2
_NS = 16
_NW = _NC * _NS
_B_PER_W = BATCH // _NW
_CHUNK = 16
_NCHUNKS = _B_PER_W // _CHUNK
_LANES = 128


def _issue_fetch(tab_hbm, stage_v, sem, i, slot):
    blk = pl.multiple_of((i // _LANES) * _LANES, _LANES)
    pltpu.async_copy(
        tab_hbm.at[:, pl.ds(blk, _LANES)], stage_v.at[slot], sem
    )


def _sc_gather(idx_hbm, tab_hbm, out_hbm, idx_v, stage_v, slab_v, sem):
    wid = lax.axis_index("s") * _NC + lax.axis_index("c")
    base = wid * _B_PER_W
    pltpu.sync_copy(idx_hbm.at[pl.ds(base, _B_PER_W)], idx_v)

    rows_lo = lax.iota(jnp.int32, 16)
    rows_hi = rows_lo + 16

    chunk0 = idx_v[pl.ds(0, _CHUNK)]
    for t in range(_CHUNK):
        _issue_fetch(tab_hbm, stage_v, sem, chunk0[t], t)

    def body(c, carry):
        cb = pl.multiple_of(c * _CHUNK, 8)
        chunk = idx_v[pl.ds(cb, _CHUNK)]
        nb = pl.multiple_of((c + 1) * _CHUNK, 8)
        nxt = idx_v[pl.ds(jnp.minimum(nb, _B_PER_W - _CHUNK), _CHUNK)]
        lanes = chunk % _LANES
        has_next = (c + 1) * _CHUNK < _B_PER_W
        for t in range(_CHUNK):
            pltpu.make_async_copy(
                tab_hbm.at[:, pl.ds(0, _LANES)], stage_v.at[t], sem
            ).wait()
            lane = jnp.full((16,), lanes[t], jnp.int32)
            col = jnp.full((16,), cb + t, jnp.int32)
            lo = plsc.load_gather(stage_v.at[t], [rows_lo, lane])
            hi = plsc.load_gather(stage_v.at[t], [rows_hi, lane])
            plsc.store_scatter(slab_v, [rows_lo, col], lo)
            plsc.store_scatter(slab_v, [rows_hi, col], hi)

            @pl.when(has_next)
            def _():
                _issue_fetch(tab_hbm, stage_v, sem, nxt[t], t)

        return carry

    lax.fori_loop(0, _NCHUNKS, body, 0)
    pltpu.sync_copy(slab_v, out_hbm.at[:, pl.ds(base, _B_PER_W)])


def kernel(num_group, table):
    mesh = plsc.VectorSubcoreMesh(core_axis_name="c", subcore_axis_name="s")
    run = functools.partial(
        pl.kernel,
        mesh=mesh,
        out_type=jax.ShapeDtypeStruct((EMBED_DIM, BATCH), jnp.float32),
        scratch_types=[
            pltpu.VMEM((_B_PER_W,), jnp.int32),
            pltpu.VMEM((_CHUNK, EMBED_DIM, _LANES), jnp.float32),
            pltpu.VMEM((EMBED_DIM, _B_PER_W), jnp.float32),
            pltpu.SemaphoreType.DMA,
        ],
        compiler_params=pltpu.CompilerParams(needs_layout_passes=False),
    )(_sc_gather)
    out_t = run(num_group.astype(jnp.int32), table.T)
    return out_t.T
